# baseline (speedup 1.0000x reference)
import functools

import jax
import jax.numpy as jnp
from jax import lax
from jax.experimental import pallas as pl
from jax.experimental.pallas import tpu as pltpu
from jax.experimental.pallas import tpu_sc as plsc

N = 10000
E = 320000
NP = 10240
EP = 344064
B = 128
K = 12
NC = 14
R = 512
NBLK = NP // R
SL = NP // 16


def _mesh():
    return plsc.VectorSubcoreMesh(core_axis_name="c", subcore_axis_name="s")


def _sc_degree(dst4, ones128, dumpidx, zeros128):
    nb = EP // 32 // B

    @functools.partial(
        pl.kernel,
        out_type=jax.ShapeDtypeStruct((2, NP, 128), jnp.float32),
        mesh=_mesh(),
        scratch_types=[
            pltpu.VMEM((nb, B), jnp.int32),
            pltpu.VMEM((B, 128), jnp.float32),
            pltpu.VMEM((B,), jnp.int32),
            pltpu.VMEM_SHARED((NP, 128), jnp.float32),
            pltpu.SemaphoreType.DMA,
            pltpu.SemaphoreType.DMA,
        ],
    )
    def k(dst_hbm, ones_hbm, dump_hbm, zeros_hbm, cnt_hbm,
          dst_all, ones_v, dump, acc, sems0, sems1):
        c = lax.axis_index("c")
        s = lax.axis_index("s")
        sems = (sems0, sems1)
        pltpu.sync_copy(ones_hbm, ones_v)
        pltpu.sync_copy(dump_hbm, dump)
        pltpu.sync_copy(dst_hbm.at[c, s], dst_all)
        pltpu.sync_copy(zeros_hbm.at[pl.ds(s * SL, SL)], acc.at[pl.ds(s * SL, SL)])
        plsc.subcore_barrier()
        pltpu.async_copy(ones_v, acc.at[dump], sems[0], add=True)
        pltpu.async_copy(ones_v, acc.at[dump], sems[1], add=True)

        def s_wait(rp):
            pltpu.make_async_copy(ones_v, acc.at[dump], sems[rp]).wait()

        def body(i, carry):
            b = 2 * i
            s_wait(0)
            pltpu.async_copy(ones_v, acc.at[dst_all.at[b]], sems[0], add=True)
            s_wait(1)
            pltpu.async_copy(ones_v, acc.at[dst_all.at[b + 1]], sems[1], add=True)
            return carry

        lax.fori_loop(0, nb // 2, body, 0)
        s_wait(0)
        s_wait(1)
        plsc.subcore_barrier()
        pltpu.sync_copy(acc.at[pl.ds(s * SL, SL)],
                        cnt_hbm.at[c, pl.ds(s * SL, SL)])

    return k(dst4, ones128, dumpidx, zeros128)


def _sc_aggregate(gcat, src5, dst4a, zeros128):
    @functools.partial(
        pl.kernel,
        out_type=jax.ShapeDtypeStruct((2, NP, 128), jnp.float32),
        mesh=_mesh(),
        scratch_types=[
            pltpu.VMEM((2, K, B), jnp.int32),
            pltpu.VMEM((2, K, B), jnp.int32),
            pltpu.VMEM((2, B, 128), jnp.float32),
            pltpu.VMEM_SHARED((NP, 128), jnp.float32),
            pltpu.SemaphoreType.DMA,
            pltpu.SemaphoreType.DMA,
            pltpu.SemaphoreType.DMA,
            pltpu.SemaphoreType.DMA,
        ],
    )
    def k(g_hbm, src_hbm, dst_hbm, z_hbm, out_hbm,
          srcc, dstc, rows, acc,
          semi0, semi1, semg0, semg1):
        c = lax.axis_index("c")
        s = lax.axis_index("s")
        semi = (semi0, semi1)
        semg = (semg0, semg1)

        def idx_load(ci, p):
            pltpu.async_copy(src_hbm.at[c, s, ci], srcc.at[p], semi[p])
            pltpu.async_copy(dst_hbm.at[s, ci], dstc.at[p], semi[p])

        def idx_wait(ci, p):
            pltpu.make_async_copy(src_hbm.at[c, s, ci], srcc.at[p], semi[p]).wait()
            pltpu.make_async_copy(dst_hbm.at[s, ci], dstc.at[p], semi[p]).wait()

        def g_fire(p, j, rp):
            pltpu.async_copy(g_hbm.at[srcc.at[p, j]], rows.at[rp], semg[rp])

        def g_wait(rp):
            pltpu.make_async_copy(g_hbm.at[srcc.at[0, 0]],
                                  rows.at[rp], semg[rp]).wait()

        def do_chunk(ci, p):
            idx_wait(ci, p)
            pltpu.async_copy(g_hbm.at[srcc.at[p, 0]], rows.at[0], semg[0])
            for j in range(K):
                rp = j % 2
                if j + 1 < K:
                    g_fire(p, j + 1, 1 - rp)
                g_wait(rp)
                pltpu.sync_copy(rows.at[rp], acc.at[dstc.at[p, j]], add=True)

        pltpu.sync_copy(z_hbm.at[pl.ds(s * SL, SL)], acc.at[pl.ds(s * SL, SL)])
        idx_load(0, 0)
        idx_load(1, 1)
        plsc.subcore_barrier()

        def body(i, carry):
            a = 2 * i
            do_chunk(a, 0)

            @pl.when(a + 2 < NC)
            def _():
                idx_load(a + 2, 0)

            do_chunk(a + 1, 1)

            @pl.when(a + 3 < NC)
            def _():
                idx_load(a + 3, 1)

            return carry

        lax.fori_loop(0, NC // 2, body, 0)
        plsc.subcore_barrier()
        pltpu.sync_copy(acc.at[pl.ds(s * SL, SL)],
                        out_hbm.at[c, pl.ds(s * SL, SL)])

    return k(gcat, src5, dst4a, zeros128)


def _dot_t(a, w):
    return lax.dot_general(a, w, (((1,), (1,)), ((), ())),
                           preferred_element_type=jnp.float32)


def _tc_layer1(x_pad, W1, c0, c1):
    def body(x_ref, w_ref, c0_ref, c1_ref, g_ref, dis_ref):
        deg = c0_ref[...] + c1_ref[...] + 1.0
        dis = lax.rsqrt(deg)
        h = _dot_t(x_ref[...], w_ref[...])
        g_ref[...] = dis * h
        dis_ref[...] = dis

    return pl.pallas_call(
        body,
        grid=(2, NBLK),
        in_specs=[
            pl.BlockSpec((R, 128), lambda i, j: (j, 0)),
            pl.BlockSpec((128, 128), lambda i, j: (i, 0)),
            pl.BlockSpec((R, 1), lambda i, j: (j, 0)),
            pl.BlockSpec((R, 1), lambda i, j: (j, 0)),
        ],
        out_specs=[
            pl.BlockSpec((R, 128), lambda i, j: (i * NBLK + j, 0)),
            pl.BlockSpec((R, 1), lambda i, j: (j, 0)),
        ],
        out_shape=[
            jax.ShapeDtypeStruct((2 * NP, 128), jnp.float32),
            jax.ShapeDtypeStruct((NP, 1), jnp.float32),
        ],
    )(x_pad, W1, c0, c1)


def _tc_mid(a0, a1, g0, g1, dis, W):
    def body(a0_ref, a1_ref, g0_ref, g1_ref, dis_ref, w_ref, out_ref):
        dis = dis_ref[...]
        t0 = jnp.maximum(dis * (a0_ref[...] + g0_ref[...]), 0.0)
        t1 = jnp.maximum(dis * (a1_ref[...] + g1_ref[...]), 0.0)
        t = jnp.concatenate([t0, t1], axis=1)
        h = _dot_t(t, w_ref[...])
        out_ref[...] = dis * h

    return pl.pallas_call(
        body,
        grid=(2, NBLK),
        in_specs=[
            pl.BlockSpec((R, 128), lambda i, j: (j, 0)),
            pl.BlockSpec((R, 128), lambda i, j: (j, 0)),
            pl.BlockSpec((R, 128), lambda i, j: (j, 0)),
            pl.BlockSpec((R, 128), lambda i, j: (j, 0)),
            pl.BlockSpec((R, 1), lambda i, j: (j, 0)),
            pl.BlockSpec((128, 256), lambda i, j: (i, 0)),
        ],
        out_specs=pl.BlockSpec((R, 128), lambda i, j: (i * NBLK + j, 0)),
        out_shape=jax.ShapeDtypeStruct((2 * NP, 128), jnp.float32),
    )(a0, a1, g0, g1, dis, W)


def _tc_final(a0, a1, g0, g1, dis):
    def body(a0_ref, a1_ref, g0_ref, g1_ref, dis_ref, out_ref):
        dis = dis_ref[...]
        o0 = dis * (a0_ref[...] + g0_ref[...])
        o1 = dis * (a1_ref[...] + g1_ref[...])
        out_ref[...] = jnp.concatenate([o0, o1], axis=1)

    return pl.pallas_call(
        body,
        grid=(NBLK,),
        in_specs=[
            pl.BlockSpec((R, 128), lambda j: (j, 0)),
            pl.BlockSpec((R, 128), lambda j: (j, 0)),
            pl.BlockSpec((R, 128), lambda j: (j, 0)),
            pl.BlockSpec((R, 128), lambda j: (j, 0)),
            pl.BlockSpec((R, 1), lambda j: (j, 0)),
        ],
        out_specs=pl.BlockSpec((R, 256), lambda j: (j, 0)),
        out_shape=jax.ShapeDtypeStruct((NP, 256), jnp.float32),
    )(a0, a1, g0, g1, dis)


def kernel(x, edge_index, W1, W2, W3):
    nbd = EP // 32 // B
    src = edge_index[0].astype(jnp.int32)
    dst = edge_index[1].astype(jnp.int32)
    pad = jnp.full((EP - E,), N, jnp.int32)
    src_p = jnp.concatenate([src, pad])
    src5 = jnp.stack([src_p, src_p + NP]).reshape(2, 16, NC, K, B)
    dst_p = jnp.concatenate([dst, pad])
    dst4a = dst_p.reshape(16, NC, K, B)
    dst4 = dst_p.reshape(2, 16, nbd, B)
    x_pad = jnp.zeros((NP, 128), jnp.float32).at[:N].set(x)
    zeros128 = jnp.zeros((NP, 128), jnp.float32)
    ones128 = jnp.ones((B, 128), jnp.float32)
    dumpidx = jnp.full((B,), N, jnp.int32)

    cnt = _sc_degree(dst4, ones128, dumpidx, zeros128)
    c0 = cnt[0, :, 0:1]
    c1 = cnt[1, :, 0:1]

    gcat1, dis = _tc_layer1(x_pad, W1, c0, c1)
    agg1 = _sc_aggregate(gcat1, src5, dst4a, zeros128)
    gcat2 = _tc_mid(agg1[0], agg1[1], gcat1[:NP], gcat1[NP:], dis, W2)
    agg2 = _sc_aggregate(gcat2, src5, dst4a, zeros128)
    gcat3 = _tc_mid(agg2[0], agg2[1], gcat2[:NP], gcat2[NP:], dis, W3)
    agg3 = _sc_aggregate(gcat3, src5, dst4a, zeros128)
    out = _tc_final(agg3[0], agg3[1], gcat3[:NP], gcat3[NP:], dis)
    return out[:N]


# speedup vs baseline: 2.0945x; 2.0945x over previous
import functools

import jax
import jax.numpy as jnp
from jax import lax
from jax.experimental import pallas as pl
from jax.experimental.pallas import tpu as pltpu
from jax.experimental.pallas import tpu_sc as plsc

N = 10000
E = 320000
NP = 10240
EP = 327680
B = 128
K = 8
NC = 20
R = 512
NBLK = NP // R
SL = NP // 16


def _mesh():
    return plsc.VectorSubcoreMesh(core_axis_name="c", subcore_axis_name="s")


def _sc_degree(dst4, ones128, dumpidx, zeros128):
    nb = EP // 32 // B

    @functools.partial(
        pl.kernel,
        out_type=jax.ShapeDtypeStruct((2, NP, 128), jnp.float32),
        mesh=_mesh(),
        scratch_types=[
            pltpu.VMEM((nb, B), jnp.int32),
            pltpu.VMEM((B, 128), jnp.float32),
            pltpu.VMEM((B,), jnp.int32),
            pltpu.VMEM_SHARED((NP, 128), jnp.float32),
            pltpu.SemaphoreType.DMA,
            pltpu.SemaphoreType.DMA,
        ],
    )
    def k(dst_hbm, ones_hbm, dump_hbm, zeros_hbm, cnt_hbm,
          dst_all, ones_v, dump, acc, sems0, sems1):
        c = lax.axis_index("c")
        s = lax.axis_index("s")
        sems = (sems0, sems1)
        pltpu.sync_copy(ones_hbm, ones_v)
        pltpu.sync_copy(dump_hbm, dump)
        pltpu.sync_copy(dst_hbm.at[c, s], dst_all)
        pltpu.sync_copy(zeros_hbm.at[pl.ds(s * SL, SL)], acc.at[pl.ds(s * SL, SL)])
        plsc.subcore_barrier()
        pltpu.async_copy(ones_v, acc.at[dump], sems[0], add=True)
        pltpu.async_copy(ones_v, acc.at[dump], sems[1], add=True)

        def s_wait(rp):
            pltpu.make_async_copy(ones_v, acc.at[dump], sems[rp]).wait()

        def body(i, carry):
            b = 2 * i
            s_wait(0)
            pltpu.async_copy(ones_v, acc.at[dst_all.at[b]], sems[0], add=True)
            s_wait(1)
            pltpu.async_copy(ones_v, acc.at[dst_all.at[b + 1]], sems[1], add=True)
            return carry

        lax.fori_loop(0, nb // 2, body, 0)
        s_wait(0)
        s_wait(1)
        plsc.subcore_barrier()
        pltpu.sync_copy(acc.at[pl.ds(s * SL, SL)],
                        cnt_hbm.at[c, pl.ds(s * SL, SL)])

    return k(dst4, ones128, dumpidx, zeros128)


def _sc_aggregate(gcat, src5, dst4a, zeros128):
    @functools.partial(
        pl.kernel,
        out_type=jax.ShapeDtypeStruct((2, NP, 128), jnp.float32),
        mesh=_mesh(),
        scratch_types=[
            pltpu.VMEM((2, K, B), jnp.int32),
            pltpu.VMEM((2, K, B), jnp.int32),
            pltpu.VMEM((2, B, 128), jnp.float32),
            pltpu.VMEM_SHARED((NP, 128), jnp.float32),
            pltpu.SemaphoreType.DMA,
            pltpu.SemaphoreType.DMA,
            pltpu.SemaphoreType.DMA,
            pltpu.SemaphoreType.DMA,
        ],
    )
    def k(g_hbm, src_hbm, dst_hbm, z_hbm, out_hbm,
          srcc, dstc, rows, acc,
          semi0, semi1, semg0, semg1):
        c = lax.axis_index("c")
        s = lax.axis_index("s")
        semi = (semi0, semi1)
        semg = (semg0, semg1)

        def idx_load(ci, p):
            pltpu.async_copy(src_hbm.at[c, s, ci], srcc.at[p], semi[p])
            pltpu.async_copy(dst_hbm.at[s, ci], dstc.at[p], semi[p])

        def idx_wait(ci, p):
            pltpu.make_async_copy(src_hbm.at[c, s, ci], srcc.at[p], semi[p]).wait()
            pltpu.make_async_copy(dst_hbm.at[s, ci], dstc.at[p], semi[p]).wait()

        def g_fire(p, j, rp):
            pltpu.async_copy(g_hbm.at[srcc.at[p, j]], rows.at[rp], semg[rp])

        def g_wait(rp):
            pltpu.make_async_copy(g_hbm.at[srcc.at[0, 0]],
                                  rows.at[rp], semg[rp]).wait()

        def do_chunk(ci, p):
            idx_wait(ci, p)
            pltpu.async_copy(g_hbm.at[srcc.at[p, 0]], rows.at[0], semg[0])
            for j in range(K):
                rp = j % 2
                if j + 1 < K:
                    g_fire(p, j + 1, 1 - rp)
                g_wait(rp)
                pltpu.sync_copy(rows.at[rp], acc.at[dstc.at[p, j]], add=True)

        pltpu.sync_copy(z_hbm.at[pl.ds(s * SL, SL)], acc.at[pl.ds(s * SL, SL)])
        idx_load(0, 0)
        idx_load(1, 1)
        plsc.subcore_barrier()

        def body(i, carry):
            a = 2 * i
            do_chunk(a, 0)

            @pl.when(a + 2 < NC)
            def _():
                idx_load(a + 2, 0)

            do_chunk(a + 1, 1)

            @pl.when(a + 3 < NC)
            def _():
                idx_load(a + 3, 1)

            return carry

        lax.fori_loop(0, NC // 2, body, 0)
        plsc.subcore_barrier()
        pltpu.sync_copy(acc.at[pl.ds(s * SL, SL)],
                        out_hbm.at[c, pl.ds(s * SL, SL)])

    return k(gcat, src5, dst4a, zeros128)


def _dot_t(a, w):
    return lax.dot_general(a, w, (((1,), (1,)), ((), ())),
                           preferred_element_type=jnp.float32)


def _tc_layer1(x_pad, W1, c0, c1):
    def body(x_ref, w_ref, c0_ref, c1_ref, g_ref, dis_ref):
        deg = c0_ref[...] + c1_ref[...] + 1.0
        dis = lax.rsqrt(deg)
        h = _dot_t(x_ref[...], w_ref[...])
        g_ref[...] = dis * h
        dis_ref[...] = dis

    return pl.pallas_call(
        body,
        grid=(2, NBLK),
        in_specs=[
            pl.BlockSpec((R, 128), lambda i, j: (j, 0)),
            pl.BlockSpec((128, 128), lambda i, j: (i, 0)),
            pl.BlockSpec((R, 1), lambda i, j: (j, 0)),
            pl.BlockSpec((R, 1), lambda i, j: (j, 0)),
        ],
        out_specs=[
            pl.BlockSpec((R, 128), lambda i, j: (i * NBLK + j, 0)),
            pl.BlockSpec((R, 1), lambda i, j: (j, 0)),
        ],
        out_shape=[
            jax.ShapeDtypeStruct((2 * NP, 128), jnp.float32),
            jax.ShapeDtypeStruct((NP, 1), jnp.float32),
        ],
    )(x_pad, W1, c0, c1)


def _tc_mid(a0, a1, g0, g1, dis, W):
    def body(a0_ref, a1_ref, g0_ref, g1_ref, dis_ref, w_ref, out_ref):
        dis = dis_ref[...]
        t0 = jnp.maximum(dis * (a0_ref[...] + g0_ref[...]), 0.0)
        t1 = jnp.maximum(dis * (a1_ref[...] + g1_ref[...]), 0.0)
        t = jnp.concatenate([t0, t1], axis=1)
        h = _dot_t(t, w_ref[...])
        out_ref[...] = dis * h

    return pl.pallas_call(
        body,
        grid=(2, NBLK),
        in_specs=[
            pl.BlockSpec((R, 128), lambda i, j: (j, 0)),
            pl.BlockSpec((R, 128), lambda i, j: (j, 0)),
            pl.BlockSpec((R, 128), lambda i, j: (j, 0)),
            pl.BlockSpec((R, 128), lambda i, j: (j, 0)),
            pl.BlockSpec((R, 1), lambda i, j: (j, 0)),
            pl.BlockSpec((128, 256), lambda i, j: (i, 0)),
        ],
        out_specs=pl.BlockSpec((R, 128), lambda i, j: (i * NBLK + j, 0)),
        out_shape=jax.ShapeDtypeStruct((2 * NP, 128), jnp.float32),
    )(a0, a1, g0, g1, dis, W)


def _tc_final(a0, a1, g0, g1, dis):
    def body(a0_ref, a1_ref, g0_ref, g1_ref, dis_ref, out_ref):
        dis = dis_ref[...]
        o0 = dis * (a0_ref[...] + g0_ref[...])
        o1 = dis * (a1_ref[...] + g1_ref[...])
        out_ref[...] = jnp.concatenate([o0, o1], axis=1)

    return pl.pallas_call(
        body,
        grid=(NBLK,),
        in_specs=[
            pl.BlockSpec((R, 128), lambda j: (j, 0)),
            pl.BlockSpec((R, 128), lambda j: (j, 0)),
            pl.BlockSpec((R, 128), lambda j: (j, 0)),
            pl.BlockSpec((R, 128), lambda j: (j, 0)),
            pl.BlockSpec((R, 1), lambda j: (j, 0)),
        ],
        out_specs=pl.BlockSpec((R, 256), lambda j: (j, 0)),
        out_shape=jax.ShapeDtypeStruct((NP, 256), jnp.float32),
    )(a0, a1, g0, g1, dis)


def kernel(x, edge_index, W1, W2, W3):
    nbd = EP // 32 // B
    src = edge_index[0].astype(jnp.int32)
    dst = edge_index[1].astype(jnp.int32)
    pad = jnp.full((EP - E,), N, jnp.int32)
    src_p = jnp.concatenate([src, pad])
    src5 = jnp.stack([src_p, src_p + NP]).reshape(2, 16, NC, K, B)
    dst_p = jnp.concatenate([dst, pad])
    dst4a = dst_p.reshape(16, NC, K, B)
    dst4 = dst_p.reshape(2, 16, nbd, B)
    x_pad = jnp.zeros((NP, 128), jnp.float32).at[:N].set(x)
    zeros128 = jnp.zeros((NP, 128), jnp.float32)
    ones128 = jnp.ones((B, 128), jnp.float32)
    dumpidx = jnp.full((B,), N, jnp.int32)

    cnt = _sc_degree(dst4, ones128, dumpidx, zeros128)
    c0 = cnt[0, :, 0:1]
    c1 = cnt[1, :, 0:1]

    gcat1, dis = _tc_layer1(x_pad, W1, c0, c1)
    agg1 = _sc_aggregate(gcat1, src5, dst4a, zeros128)
    gcat2 = _tc_mid(agg1[0], agg1[1], gcat1[:NP], gcat1[NP:], dis, W2)
    agg2 = _sc_aggregate(gcat2, src5, dst4a, zeros128)
    gcat3 = _tc_mid(agg2[0], agg2[1], gcat2[:NP], gcat2[NP:], dis, W3)
    agg3 = _sc_aggregate(gcat3, src5, dst4a, zeros128)
    out = _tc_final(agg3[0], agg3[1], gcat3[:NP], gcat3[NP:], dis)
    return out[:N]
